# baseline (speedup 1.0000x reference)
import functools

import jax
import jax.numpy as jnp
from jax import lax
from jax.experimental import pallas as pl
from jax.experimental.pallas import tpu as pltpu
from jax.experimental.pallas import tpu_sc as plsc

N_TOTAL = 8_000_000
NBINS = 10
LANES = 16
NWORKERS = 32
TILE = 128

BLK = 131072
R = BLK // TILE
TC_BLOCKS = 21
TC_N = TC_BLOCKS * BLK

MAIN = (N_TOTAL // (TILE * NWORKERS)) * TILE * NWORKERS
PER_W = (MAIN - TC_N) // NWORKERS
CHUNK = 183 * TILE
NCHUNK = PER_W // CHUNK
UNROLL = 8
TAIL_OFF = MAIN
TAIL = N_TOTAL - TAIL_OFF


def _sc_pass(pred, target):
    mesh = plsc.VectorSubcoreMesh(core_axis_name="c", subcore_axis_name="s")

    @functools.partial(
        pl.kernel,
        out_type=(
            jax.ShapeDtypeStruct((NWORKERS, NBINS, LANES), jnp.float32),
            jax.ShapeDtypeStruct((NWORKERS, NBINS, LANES), jnp.float32),
        ),
        name="ghmr_sc_pass",
        mesh=mesh,
        scratch_types=[
            pltpu.VMEM((CHUNK,), jnp.float32),
            pltpu.VMEM((CHUNK,), jnp.float32),
            pltpu.VMEM((CHUNK,), jnp.float32),
            pltpu.VMEM((CHUNK,), jnp.float32),
            pltpu.VMEM((NBINS, LANES), jnp.float32),
            pltpu.VMEM((NBINS, LANES), jnp.float32),
            pltpu.SemaphoreType.DMA,
            pltpu.SemaphoreType.DMA,
            pltpu.SemaphoreType.DMA,
            pltpu.SemaphoreType.DMA,
        ],
        compiler_params=pltpu.CompilerParams(needs_layout_passes=False),
    )
    def sc_kernel(pred_hbm, target_hbm, cnt_hbm, sum_hbm, pbuf0, pbuf1, tbuf0, tbuf1, acc_cnt, acc_sum, sp0, sp1, st0, st1):
        pbufs = [pbuf0, pbuf1]
        tbufs = [tbuf0, tbuf1]
        wid = lax.axis_index("s") * 2 + lax.axis_index("c")
        base = TC_N + wid * PER_W
        for r in range(NBINS):
            acc_cnt[r, :] = jnp.zeros((LANES,), jnp.float32)
            acc_sum[r, :] = jnp.zeros((LANES,), jnp.float32)
        lane = lax.iota(jnp.int32, LANES)
        ones = jnp.ones((LANES,), jnp.float32)
        psem = [sp0, sp1]
        tsem = [st0, st1]

        def start(c, slot):
            off = pl.multiple_of(base + c * CHUNK, TILE)
            pltpu.make_async_copy(pred_hbm.at[0, pl.ds(off, CHUNK)], pbufs[slot], psem[slot]).start()
            pltpu.make_async_copy(target_hbm.at[0, pl.ds(off, CHUNK)], tbufs[slot], tsem[slot]).start()

        def wait(slot):
            pltpu.make_async_copy(pred_hbm.at[0, pl.ds(base, CHUNK)], pbufs[slot], psem[slot]).wait()
            pltpu.make_async_copy(target_hbm.at[0, pl.ds(base, CHUNK)], tbufs[slot], tsem[slot]).wait()

        def vec_body(pbuf, tbuf, off):
            p = pbuf[pl.ds(off, LANES)]
            t = tbuf[pl.ds(off, LANES)]
            d = p - t
            ad = jnp.abs(d)
            diff = jnp.minimum(ad, jnp.float32(360.0) - ad)
            bf = diff * jnp.float32(10.0 / 180.0)
            b = jnp.minimum(bf.astype(jnp.int32), 9)
            m = jnp.minimum(ad, jnp.float32(1.0))
            loss = jnp.float32(0.5) * m * m + (ad - m)
            plsc.addupdate_scatter(acc_cnt, [b, lane], ones)
            plsc.addupdate_scatter(acc_sum, [b, lane], loss)

        start(0, 0)
        for c in range(NCHUNK):
            slot = c & 1
            if c + 1 < NCHUNK:
                start(c + 1, slot ^ 1)
            wait(slot)

            @plsc.parallel_loop(0, CHUNK, step=LANES, unroll=UNROLL)
            def body(off):
                vec_body(pbufs[slot], tbufs[slot], pl.multiple_of(off, LANES))

        @pl.when(wid == 0)
        def _tail():
            pltpu.make_async_copy(
                pred_hbm.at[0, pl.ds(TAIL_OFF, TAIL)], pbufs[0].at[pl.ds(0, TAIL)], psem[0]
            ).start()
            pltpu.make_async_copy(
                target_hbm.at[0, pl.ds(TAIL_OFF, TAIL)], tbufs[0].at[pl.ds(0, TAIL)], tsem[0]
            ).start()
            pltpu.make_async_copy(
                pred_hbm.at[0, pl.ds(TAIL_OFF, TAIL)], pbufs[0].at[pl.ds(0, TAIL)], psem[0]
            ).wait()
            pltpu.make_async_copy(
                target_hbm.at[0, pl.ds(TAIL_OFF, TAIL)], tbufs[0].at[pl.ds(0, TAIL)], tsem[0]
            ).wait()

            @plsc.parallel_loop(0, TAIL, step=LANES, unroll=8)
            def tail_body(off):
                vec_body(pbufs[0], tbufs[0], pl.multiple_of(off, LANES))

        pltpu.sync_copy(acc_cnt, cnt_hbm.at[wid])
        pltpu.sync_copy(acc_sum, sum_hbm.at[wid])

    return sc_kernel(pred, target)


def _tc_pass(pred, target):
    def body(p_ref, t_ref, o_ref):
        i = pl.program_id(0)
        p = p_ref[...].reshape(R, TILE)
        t = t_ref[...].reshape(R, TILE)
        d = p - t
        ad = jnp.abs(d)
        diff = jnp.minimum(ad, jnp.float32(360.0) - ad)
        bf = diff * jnp.float32(10.0 / 180.0)
        m = jnp.minimum(ad, jnp.float32(1.0))
        loss = jnp.float32(0.5) * m * m + (ad - m)
        zero = jnp.zeros((1, TILE), jnp.float32)
        crows = [zero]
        srows = [jnp.sum(loss, axis=0, keepdims=True)]
        for k in range(1, NBINS):
            mask_f = (bf >= jnp.float32(k)).astype(jnp.float32)
            crows.append(jnp.sum(mask_f, axis=0, keepdims=True))
            srows.append(jnp.sum(mask_f * loss, axis=0, keepdims=True))
        upd = jnp.concatenate(crows + srows, axis=0)

        @pl.when(i == 0)
        def _init():
            o_ref[...] = jnp.zeros((2 * NBINS, TILE), jnp.float32)

        o_ref[...] = o_ref[...] + upd

    return pl.pallas_call(
        body,
        grid=(TC_BLOCKS,),
        in_specs=[
            pl.BlockSpec((1, BLK), lambda i: (0, i)),
            pl.BlockSpec((1, BLK), lambda i: (0, i)),
        ],
        out_specs=pl.BlockSpec((2 * NBINS, TILE), lambda i: (0, 0)),
        out_shape=jax.ShapeDtypeStruct((2 * NBINS, TILE), jnp.float32),
    )(pred, target)


def _combine(sc_cnt, sc_sum, tc_acc):
    def ck(xc_ref, xs_ref, tc_ref, o_ref):
        total = jnp.float32(N_TOTAL)
        counts_sc = jnp.sum(jnp.sum(xc_ref[...], axis=0), axis=1, keepdims=True)
        sums_sc = jnp.sum(jnp.sum(xs_ref[...], axis=0), axis=1, keepdims=True)
        tcs = jnp.sum(tc_ref[...], axis=1, keepdims=True)
        ccum = tcs[0:NBINS]
        scum = tcs[NBINS : 2 * NBINS]
        zero1 = jnp.zeros((1, 1), jnp.float32)
        cc = jnp.concatenate([jnp.full((1, 1), jnp.float32(TC_N)), ccum[1:NBINS]], axis=0)
        cnext = jnp.concatenate([ccum[1:NBINS], zero1], axis=0)
        counts_tc = cc - cnext
        snext = jnp.concatenate([scum[1:NBINS], zero1], axis=0)
        sums_tc = scum - snext
        counts = counts_sc + counts_tc
        sums = sums_sc + sums_tc
        accm = jnp.where(counts > 0, jnp.float32(0.1) * counts, jnp.float32(0.0))
        n = jnp.sum((counts > 0).astype(jnp.float32))
        n_safe = jnp.maximum(n, jnp.float32(1.0))
        w = jnp.where(
            accm > 0,
            total / (n_safe * jnp.maximum(accm, jnp.float32(1e-12))),
            jnp.float32(0.0),
        )
        walpha = jnp.where(
            w > 0,
            jnp.exp(jnp.float32(0.75) * jnp.log(jnp.maximum(w, jnp.float32(1e-30)))),
            jnp.float32(0.0),
        )
        tot = jnp.sum(jnp.where(counts > 0, walpha * sums, jnp.float32(0.0)))
        tot = jnp.where(n > 0, tot, jnp.sum(sums))
        o_ref[...] = jnp.reshape(tot / total, (1, 1))

    return pl.pallas_call(ck, out_shape=jax.ShapeDtypeStruct((1, 1), jnp.float32))(
        sc_cnt, sc_sum, tc_acc
    )


def kernel(pred, target):
    p = pred.T
    t = target.T
    cnt, sm = _sc_pass(p, t)
    tc_acc = _tc_pass(p, t)
    return _combine(cnt, sm, tc_acc)[0, 0]


# speedup vs baseline: 1.0227x; 1.0227x over previous
import functools

import jax
import jax.numpy as jnp
from jax import lax
from jax.experimental import pallas as pl
from jax.experimental.pallas import tpu as pltpu
from jax.experimental.pallas import tpu_sc as plsc

N_TOTAL = 8_000_000
NBINS = 10
LANES = 16
NWORKERS = 32
TILE = 128

BLK = 65536
R = BLK // TILE
TC_BLOCKS = 43
TC_N = TC_BLOCKS * BLK

MAIN = (N_TOTAL // (TILE * NWORKERS)) * TILE * NWORKERS
PER_W = (MAIN - TC_N) // NWORKERS
CHUNK = 115 * TILE
NCHUNK = PER_W // CHUNK
UNROLL = 8
TAIL_OFF = MAIN
TAIL = N_TOTAL - TAIL_OFF


def _sc_pass(pred, target):
    mesh = plsc.VectorSubcoreMesh(core_axis_name="c", subcore_axis_name="s")

    @functools.partial(
        pl.kernel,
        out_type=(
            jax.ShapeDtypeStruct((NWORKERS, NBINS, LANES), jnp.float32),
            jax.ShapeDtypeStruct((NWORKERS, NBINS, LANES), jnp.float32),
        ),
        name="ghmr_sc_pass",
        mesh=mesh,
        scratch_types=[
            pltpu.VMEM((CHUNK,), jnp.float32),
            pltpu.VMEM((CHUNK,), jnp.float32),
            pltpu.VMEM((CHUNK,), jnp.float32),
            pltpu.VMEM((CHUNK,), jnp.float32),
            pltpu.VMEM((NBINS, LANES), jnp.float32),
            pltpu.VMEM((NBINS, LANES), jnp.float32),
            pltpu.SemaphoreType.DMA,
            pltpu.SemaphoreType.DMA,
            pltpu.SemaphoreType.DMA,
            pltpu.SemaphoreType.DMA,
        ],
        compiler_params=pltpu.CompilerParams(needs_layout_passes=False),
    )
    def sc_kernel(pred_hbm, target_hbm, cnt_hbm, sum_hbm, pbuf0, pbuf1, tbuf0, tbuf1, acc_cnt, acc_sum, sp0, sp1, st0, st1):
        pbufs = [pbuf0, pbuf1]
        tbufs = [tbuf0, tbuf1]
        wid = lax.axis_index("s") * 2 + lax.axis_index("c")
        base = TC_N + wid * PER_W
        for r in range(NBINS):
            acc_cnt[r, :] = jnp.zeros((LANES,), jnp.float32)
            acc_sum[r, :] = jnp.zeros((LANES,), jnp.float32)
        lane = lax.iota(jnp.int32, LANES)
        ones = jnp.ones((LANES,), jnp.float32)
        psem = [sp0, sp1]
        tsem = [st0, st1]

        def start(c, slot):
            off = pl.multiple_of(base + c * CHUNK, TILE)
            pltpu.make_async_copy(pred_hbm.at[0, pl.ds(off, CHUNK)], pbufs[slot], psem[slot]).start()
            pltpu.make_async_copy(target_hbm.at[0, pl.ds(off, CHUNK)], tbufs[slot], tsem[slot]).start()

        def wait(slot):
            pltpu.make_async_copy(pred_hbm.at[0, pl.ds(base, CHUNK)], pbufs[slot], psem[slot]).wait()
            pltpu.make_async_copy(target_hbm.at[0, pl.ds(base, CHUNK)], tbufs[slot], tsem[slot]).wait()

        def vec_body(pbuf, tbuf, off):
            p = pbuf[pl.ds(off, LANES)]
            t = tbuf[pl.ds(off, LANES)]
            d = p - t
            ad = jnp.abs(d)
            diff = jnp.minimum(ad, jnp.float32(360.0) - ad)
            bf = diff * jnp.float32(10.0 / 180.0)
            b = jnp.minimum(bf.astype(jnp.int32), 9)
            m = jnp.minimum(ad, jnp.float32(1.0))
            loss = jnp.float32(0.5) * m * m + (ad - m)
            plsc.addupdate_scatter(acc_cnt, [b, lane], ones)
            plsc.addupdate_scatter(acc_sum, [b, lane], loss)

        start(0, 0)
        for c in range(NCHUNK):
            slot = c & 1
            if c + 1 < NCHUNK:
                start(c + 1, slot ^ 1)
            wait(slot)

            @plsc.parallel_loop(0, CHUNK, step=LANES, unroll=UNROLL)
            def body(off):
                vec_body(pbufs[slot], tbufs[slot], pl.multiple_of(off, LANES))

        @pl.when(wid == 0)
        def _tail():
            pltpu.make_async_copy(
                pred_hbm.at[0, pl.ds(TAIL_OFF, TAIL)], pbufs[0].at[pl.ds(0, TAIL)], psem[0]
            ).start()
            pltpu.make_async_copy(
                target_hbm.at[0, pl.ds(TAIL_OFF, TAIL)], tbufs[0].at[pl.ds(0, TAIL)], tsem[0]
            ).start()
            pltpu.make_async_copy(
                pred_hbm.at[0, pl.ds(TAIL_OFF, TAIL)], pbufs[0].at[pl.ds(0, TAIL)], psem[0]
            ).wait()
            pltpu.make_async_copy(
                target_hbm.at[0, pl.ds(TAIL_OFF, TAIL)], tbufs[0].at[pl.ds(0, TAIL)], tsem[0]
            ).wait()

            @plsc.parallel_loop(0, TAIL, step=LANES, unroll=8)
            def tail_body(off):
                vec_body(pbufs[0], tbufs[0], pl.multiple_of(off, LANES))

        pltpu.sync_copy(acc_cnt, cnt_hbm.at[wid])
        pltpu.sync_copy(acc_sum, sum_hbm.at[wid])

    return sc_kernel(pred, target)


def _tc_pass(pred, target):
    def body(p_ref, t_ref, o_ref):
        i = pl.program_id(0)
        p = p_ref[...].reshape(R, TILE)
        t = t_ref[...].reshape(R, TILE)
        d = p - t
        ad = jnp.abs(d)
        diff = jnp.minimum(ad, jnp.float32(360.0) - ad)
        bf = diff * jnp.float32(10.0 / 180.0)
        m = jnp.minimum(ad, jnp.float32(1.0))
        loss = jnp.float32(0.5) * m * m + (ad - m)
        zero = jnp.zeros((1, TILE), jnp.float32)
        crows = [zero]
        srows = [jnp.sum(loss, axis=0, keepdims=True)]
        for k in range(1, NBINS):
            mask_f = (bf >= jnp.float32(k)).astype(jnp.float32)
            crows.append(jnp.sum(mask_f, axis=0, keepdims=True))
            srows.append(jnp.sum(mask_f * loss, axis=0, keepdims=True))
        upd = jnp.concatenate(crows + srows, axis=0)

        @pl.when(i == 0)
        def _init():
            o_ref[...] = jnp.zeros((2 * NBINS, TILE), jnp.float32)

        o_ref[...] = o_ref[...] + upd

    return pl.pallas_call(
        body,
        grid=(TC_BLOCKS,),
        in_specs=[
            pl.BlockSpec((1, BLK), lambda i: (0, i)),
            pl.BlockSpec((1, BLK), lambda i: (0, i)),
        ],
        out_specs=pl.BlockSpec((2 * NBINS, TILE), lambda i: (0, 0)),
        out_shape=jax.ShapeDtypeStruct((2 * NBINS, TILE), jnp.float32),
    )(pred, target)


def _combine(sc_cnt, sc_sum, tc_acc):
    def ck(xc_ref, xs_ref, tc_ref, o_ref):
        total = jnp.float32(N_TOTAL)
        counts_sc = jnp.sum(jnp.sum(xc_ref[...], axis=0), axis=1, keepdims=True)
        sums_sc = jnp.sum(jnp.sum(xs_ref[...], axis=0), axis=1, keepdims=True)
        tcs = jnp.sum(tc_ref[...], axis=1, keepdims=True)
        ccum = tcs[0:NBINS]
        scum = tcs[NBINS : 2 * NBINS]
        zero1 = jnp.zeros((1, 1), jnp.float32)
        cc = jnp.concatenate([jnp.full((1, 1), jnp.float32(TC_N)), ccum[1:NBINS]], axis=0)
        cnext = jnp.concatenate([ccum[1:NBINS], zero1], axis=0)
        counts_tc = cc - cnext
        snext = jnp.concatenate([scum[1:NBINS], zero1], axis=0)
        sums_tc = scum - snext
        counts = counts_sc + counts_tc
        sums = sums_sc + sums_tc
        accm = jnp.where(counts > 0, jnp.float32(0.1) * counts, jnp.float32(0.0))
        n = jnp.sum((counts > 0).astype(jnp.float32))
        n_safe = jnp.maximum(n, jnp.float32(1.0))
        w = jnp.where(
            accm > 0,
            total / (n_safe * jnp.maximum(accm, jnp.float32(1e-12))),
            jnp.float32(0.0),
        )
        walpha = jnp.where(
            w > 0,
            jnp.exp(jnp.float32(0.75) * jnp.log(jnp.maximum(w, jnp.float32(1e-30)))),
            jnp.float32(0.0),
        )
        tot = jnp.sum(jnp.where(counts > 0, walpha * sums, jnp.float32(0.0)))
        tot = jnp.where(n > 0, tot, jnp.sum(sums))
        o_ref[...] = jnp.reshape(tot / total, (1, 1))

    return pl.pallas_call(ck, out_shape=jax.ShapeDtypeStruct((1, 1), jnp.float32))(
        sc_cnt, sc_sum, tc_acc
    )


def kernel(pred, target):
    p = pred.T
    t = target.T
    cnt, sm = _sc_pass(p, t)
    tc_acc = _tc_pass(p, t)
    return _combine(cnt, sm, tc_acc)[0, 0]
